# baseline (speedup 1.0000x reference)
import functools

import jax
import jax.numpy as jnp
from jax import lax
from jax.experimental import pallas as pl
from jax.experimental.pallas import tpu as pltpu
from jax.experimental.pallas import tpu_sc as plsc

NC = 2
NS = 16
NW = NC * NS
CHUNK = 80


def _segsum_body(n, ew, nch,
                 edge_hbm, recv_hbm, zeros_hbm, out_hbm,
                 idx0, idx1, rows0, rows1, acc_sh,
                 lsem0, lsem1, ssem0, ssem1):
    c = lax.axis_index("c")
    s = lax.axis_index("s")
    wid = c * NS + s

    @pl.when(s == 0)
    def _():
        pltpu.sync_copy(zeros_hbm, acc_sh)

    plsc.subcore_barrier()
    base_e = wid * ew

    def off(i):
        return pl.multiple_of(base_e + i * CHUNK, CHUNK)

    def load(i, idx_v, rows_v, lsem):
        b = off(i)
        pltpu.async_copy(recv_hbm.at[pl.ds(b, CHUNK)], idx_v, lsem)
        pltpu.async_copy(edge_hbm.at[pl.ds(b, CHUNK)], rows_v, lsem)

    def wait_load(idx_v, rows_v, lsem):
        pltpu.make_async_copy(recv_hbm.at[pl.ds(0, CHUNK)], idx_v, lsem).wait()
        pltpu.make_async_copy(edge_hbm.at[pl.ds(0, CHUNK)], rows_v, lsem).wait()

    def scatter(idx_v, rows_v, ssem):
        pltpu.async_copy(rows_v, acc_sh.at[idx_v], ssem, add=True)

    def wait_scatter(idx_v, rows_v, ssem):
        pltpu.make_async_copy(rows_v, acc_sh.at[idx_v], ssem).wait()

    load(0, idx0, rows0, lsem0)

    def pair(j, carry):
        @pl.when(j > 0)
        def _():
            wait_scatter(idx1, rows1, ssem1)

        load(2 * j + 1, idx1, rows1, lsem1)
        wait_load(idx0, rows0, lsem0)
        scatter(idx0, rows0, ssem0)

        @pl.when(2 * j + 2 < nch)
        def _():
            wait_scatter(idx0, rows0, ssem0)
            load(2 * j + 2, idx0, rows0, lsem0)

        wait_load(idx1, rows1, lsem1)
        scatter(idx1, rows1, ssem1)
        return carry

    lax.fori_loop(0, nch // 2, pair, 0)
    if nch % 2:
        wait_load(idx0, rows0, lsem0)
        scatter(idx0, rows0, ssem0)
        wait_scatter(idx0, rows0, ssem0)
    wait_scatter(idx1, rows1, ssem1)
    plsc.subcore_barrier()

    @pl.when(s == 0)
    def _():
        pltpu.sync_copy(acc_sh, out_hbm.at[pl.ds(pl.multiple_of(c * n, 8), n)])


def _segsum(edge_attr, receivers, zeros):
    e, h = edge_attr.shape
    n = zeros.shape[0]
    ew = e // NW
    nch = ew // CHUNK
    mesh = plsc.VectorSubcoreMesh(core_axis_name="c", subcore_axis_name="s")
    k = pl.kernel(
        functools.partial(_segsum_body, n, ew, nch),
        out_type=jax.ShapeDtypeStruct((NC * n, h), jnp.float32),
        mesh=mesh,
        scratch_types=[
            pltpu.VMEM((CHUNK,), jnp.int32),
            pltpu.VMEM((CHUNK,), jnp.int32),
            pltpu.VMEM((CHUNK, h), jnp.float32),
            pltpu.VMEM((CHUNK, h), jnp.float32),
            pltpu.VMEM_SHARED((n, h), jnp.float32),
            pltpu.SemaphoreType.DMA,
            pltpu.SemaphoreType.DMA,
            pltpu.SemaphoreType.DMA,
            pltpu.SemaphoreType.DMA,
        ],
    )
    return k(edge_attr, receivers, zeros)


def _gather_body(ew, nch,
                 xa_hbm, xb_hbm, snd_hbm, rcv_hbm, g_hbm,
                 idxs0, idxr0, idxs1, idxr1, bufa0, bufb0, bufa1, bufb1,
                 gsem0, gsem1, wsem0, wsem1):
    c = lax.axis_index("c")
    s = lax.axis_index("s")
    wid = c * NS + s
    base_e = wid * ew

    def off(i):
        return pl.multiple_of(base_e + i * CHUNK, CHUNK)

    def issue(i, idxs_v, idxr_v, ba, bb, gsem):
        b = off(i)
        pltpu.sync_copy(snd_hbm.at[pl.ds(b, CHUNK)], idxs_v)
        pltpu.sync_copy(rcv_hbm.at[pl.ds(b, CHUNK)], idxr_v)
        pltpu.async_copy(xa_hbm.at[idxs_v], ba, gsem)
        pltpu.async_copy(xb_hbm.at[idxr_v], bb, gsem)

    def wait_gather(idxs_v, idxr_v, ba, bb, gsem):
        pltpu.make_async_copy(xa_hbm.at[idxs_v], ba, gsem).wait()
        pltpu.make_async_copy(xb_hbm.at[idxr_v], bb, gsem).wait()

    def add_wb(i, ba, bb, wsem):
        def row(j, carry2):
            for kk in range(8):
                plsc.addupdate(ba.at[j, pl.ds(kk * 16, 16)],
                               bb[j, pl.ds(kk * 16, 16)])
            return carry2

        lax.fori_loop(0, CHUNK, row, 0)
        pltpu.async_copy(ba, g_hbm.at[pl.ds(off(i), CHUNK)], wsem)

    def wait_wb(ba, wsem):
        pltpu.make_async_copy(ba, g_hbm.at[pl.ds(0, CHUNK)], wsem).wait()

    issue(0, idxs0, idxr0, bufa0, bufb0, gsem0)

    def pair(j, carry):
        @pl.when(j > 0)
        def _():
            wait_wb(bufa1, wsem1)

        issue(2 * j + 1, idxs1, idxr1, bufa1, bufb1, gsem1)
        wait_gather(idxs0, idxr0, bufa0, bufb0, gsem0)
        add_wb(2 * j, bufa0, bufb0, wsem0)

        @pl.when(2 * j + 2 < nch)
        def _():
            wait_wb(bufa0, wsem0)
            issue(2 * j + 2, idxs0, idxr0, bufa0, bufb0, gsem0)

        wait_gather(idxs1, idxr1, bufa1, bufb1, gsem1)
        add_wb(2 * j + 1, bufa1, bufb1, wsem1)
        return carry

    lax.fori_loop(0, nch // 2, pair, 0)
    if nch % 2:
        wait_gather(idxs0, idxr0, bufa0, bufb0, gsem0)
        add_wb(nch - 1, bufa0, bufb0, wsem0)
        wait_wb(bufa0, wsem0)
    wait_wb(bufa1, wsem1)


def _gather_add(xa, xb, senders, receivers):
    n, h = xa.shape
    e = senders.shape[0]
    ew = e // NW
    nch = ew // CHUNK
    mesh = plsc.VectorSubcoreMesh(core_axis_name="c", subcore_axis_name="s")
    k = pl.kernel(
        functools.partial(_gather_body, ew, nch),
        out_type=jax.ShapeDtypeStruct((e, h), jnp.float32),
        mesh=mesh,
        scratch_types=[
            pltpu.VMEM((CHUNK,), jnp.int32),
            pltpu.VMEM((CHUNK,), jnp.int32),
            pltpu.VMEM((CHUNK,), jnp.int32),
            pltpu.VMEM((CHUNK,), jnp.int32),
            pltpu.VMEM((CHUNK, h), jnp.float32),
            pltpu.VMEM((CHUNK, h), jnp.float32),
            pltpu.VMEM((CHUNK, h), jnp.float32),
            pltpu.VMEM((CHUNK, h), jnp.float32),
            pltpu.SemaphoreType.DMA,
            pltpu.SemaphoreType.DMA,
            pltpu.SemaphoreType.DMA,
            pltpu.SemaphoreType.DMA,
        ],
    )
    return k(xa, xb, senders, receivers)


def _layer_norm(hh, gamma, beta):
    mu = jnp.mean(hh, axis=-1, keepdims=True)
    var = jnp.mean((hh - mu) ** 2, axis=-1, keepdims=True)
    return (hh - mu) * lax.rsqrt(var + 1e-5) * gamma + beta


def _cell_body(n, x_ref, na_ref, aggp_ref,
               w1a_ref, w1b_ref, b1_ref, w2_ref, b2_ref, w3_ref, b3_ref,
               g_ref, bt_ref, ew1a_ref, ew1b_ref,
               xout_ref, xa_ref, xb_ref):
    f32 = jnp.float32
    agg = aggp_ref[:n, :] + aggp_ref[n:, :]
    nb = jnp.dot(na_ref[...], w1b_ref[...], preferred_element_type=f32) + b1_ref[...]

    def mlp(xin):
        hh = jax.nn.silu(
            jnp.dot(xin + agg, w1a_ref[...], preferred_element_type=f32) + nb)
        hh = jax.nn.silu(
            jnp.dot(hh, w2_ref[...], preferred_element_type=f32) + b2_ref[...])
        hh = jnp.dot(hh, w3_ref[...], preferred_element_type=f32) + b3_ref[...]
        return _layer_norm(hh, g_ref[...], bt_ref[...])

    x0 = x_ref[...]
    x2 = mlp(mlp(x0))
    xout_ref[...] = x0 + x2
    xa_ref[...] = jnp.dot(x2, ew1a_ref[...], preferred_element_type=f32)
    xb_ref[...] = jnp.dot(x2, ew1b_ref[...], preferred_element_type=f32)


def _cell(x, node_attr, aggp, cb_W1, cb_b1, cb_W2, cb_b2, cb_W3, cb_b3,
          cb_g, cb_bt, eb_W1):
    n, h = x.shape
    w1a, w1b = cb_W1[:h], cb_W1[h:]
    ew1a, ew1b = eb_W1[:h], eb_W1[h:2 * h]
    row = lambda v: v.reshape(1, h)
    out = pl.pallas_call(
        functools.partial(_cell_body, n),
        out_shape=[jax.ShapeDtypeStruct((n, h), jnp.float32)] * 3,
    )(x, node_attr, aggp, w1a, w1b, row(cb_b1), cb_W2, row(cb_b2),
      cb_W3, row(cb_b3), row(cb_g), row(cb_bt), ew1a, ew1b)
    return out


def _edge_pre_body(e_ref, w1c_ref, b1_ref, out_ref):
    out_ref[...] = (jnp.dot(e_ref[...], w1c_ref[...],
                            preferred_element_type=jnp.float32) + b1_ref[...])


def _edge_pre(edge_attr, eb_W1, eb_b1):
    e, h = edge_attr.shape
    r = 4000
    blk = pl.BlockSpec((r, h), lambda i: (i, 0))
    return pl.pallas_call(
        _edge_pre_body,
        grid=(e // r,),
        in_specs=[blk, pl.BlockSpec((h, h), lambda i: (0, 0)),
                  pl.BlockSpec((1, h), lambda i: (0, 0))],
        out_specs=blk,
        out_shape=jax.ShapeDtypeStruct((e, h), jnp.float32),
    )(edge_attr, eb_W1[2 * h:], eb_b1.reshape(1, h))


def _edge_body(g_ref, t_ref, e_ref, w2_ref, b2_ref, w3_ref, b3_ref,
               gm_ref, bt_ref, out_ref):
    f32 = jnp.float32
    hh = jax.nn.silu(g_ref[...] + t_ref[...])
    hh = jax.nn.silu(
        jnp.dot(hh, w2_ref[...], preferred_element_type=f32) + b2_ref[...])
    hh = jnp.dot(hh, w3_ref[...], preferred_element_type=f32) + b3_ref[...]
    out_ref[...] = e_ref[...] + _layer_norm(hh, gm_ref[...], bt_ref[...])


def _edge(g, t, edge_attr, eb_W2, eb_b2, eb_W3, eb_b3, eb_g, eb_bt):
    e, h = edge_attr.shape
    r = 4000
    row = lambda v: v.reshape(1, h)
    blk = pl.BlockSpec((r, h), lambda i: (i, 0))
    wspec = pl.BlockSpec((h, h), lambda i: (0, 0))
    bspec = pl.BlockSpec((1, h), lambda i: (0, 0))
    return pl.pallas_call(
        _edge_body,
        grid=(e // r,),
        in_specs=[blk, blk, blk, wspec, bspec, wspec, bspec, bspec, bspec],
        out_specs=blk,
        out_shape=jax.ShapeDtypeStruct((e, h), jnp.float32),
    )(g, t, edge_attr, eb_W2, row(eb_b2), eb_W3, row(eb_b3),
      row(eb_g), row(eb_bt))


def kernel(x, edge_attr, node_attr, edge_index,
           cb_W1, cb_b1, cb_W2, cb_b2, cb_W3, cb_b3, cb_g, cb_bt,
           eb_W1, eb_b1, eb_W2, eb_b2, eb_W3, eb_b3, eb_g, eb_bt):
    senders = edge_index[0]
    receivers = edge_index[1]
    n, h = x.shape
    zeros = jnp.zeros((n, h), jnp.float32)
    aggp = _segsum(edge_attr, receivers, zeros)
    x_out, xa, xb = _cell(x, node_attr, aggp, cb_W1, cb_b1, cb_W2, cb_b2,
                          cb_W3, cb_b3, cb_g, cb_bt, eb_W1)
    g = _gather_add(xa, xb, senders, receivers)
    t = _edge_pre(edge_attr, eb_W1, eb_b1)
    e_out = _edge(g, t, edge_attr, eb_W2, eb_b2, eb_W3, eb_b3, eb_g, eb_bt)
    return (x_out, e_out)


# speedup vs baseline: 1.3094x; 1.3094x over previous
import functools

import jax
import jax.numpy as jnp
from jax import lax
from jax.experimental import pallas as pl
from jax.experimental.pallas import tpu as pltpu
from jax.experimental.pallas import tpu_sc as plsc

NC = 2
NS = 16
NW = NC * NS
CHUNK = 80


def _segsum_body(n, ew, nch,
                 edge_hbm, recv_hbm, zeros_hbm, out_hbm,
                 idx0, idx1, rows0, rows1, acc_sh,
                 lsem0, lsem1, ssem0, ssem1):
    c = lax.axis_index("c")
    s = lax.axis_index("s")
    wid = c * NS + s

    @pl.when(s == 0)
    def _():
        pltpu.sync_copy(zeros_hbm, acc_sh)

    plsc.subcore_barrier()
    base_e = wid * ew

    def off(i):
        return pl.multiple_of(base_e + i * CHUNK, CHUNK)

    def load(i, idx_v, rows_v, lsem):
        b = off(i)
        pltpu.async_copy(recv_hbm.at[pl.ds(b, CHUNK)], idx_v, lsem)
        pltpu.async_copy(edge_hbm.at[pl.ds(b, CHUNK)], rows_v, lsem)

    def wait_load(idx_v, rows_v, lsem):
        pltpu.make_async_copy(recv_hbm.at[pl.ds(0, CHUNK)], idx_v, lsem).wait()
        pltpu.make_async_copy(edge_hbm.at[pl.ds(0, CHUNK)], rows_v, lsem).wait()

    def scatter(idx_v, rows_v, ssem):
        pltpu.async_copy(rows_v, acc_sh.at[idx_v], ssem, add=True)

    def wait_scatter(idx_v, rows_v, ssem):
        pltpu.make_async_copy(rows_v, acc_sh.at[idx_v], ssem).wait()

    load(0, idx0, rows0, lsem0)

    def pair(j, carry):
        @pl.when(j > 0)
        def _():
            wait_scatter(idx1, rows1, ssem1)

        load(2 * j + 1, idx1, rows1, lsem1)
        wait_load(idx0, rows0, lsem0)
        scatter(idx0, rows0, ssem0)

        @pl.when(2 * j + 2 < nch)
        def _():
            wait_scatter(idx0, rows0, ssem0)
            load(2 * j + 2, idx0, rows0, lsem0)

        wait_load(idx1, rows1, lsem1)
        scatter(idx1, rows1, ssem1)
        return carry

    lax.fori_loop(0, nch // 2, pair, 0)
    if nch % 2:
        wait_load(idx0, rows0, lsem0)
        scatter(idx0, rows0, ssem0)
        wait_scatter(idx0, rows0, ssem0)
    wait_scatter(idx1, rows1, ssem1)
    plsc.subcore_barrier()

    @pl.when(s == 0)
    def _():
        pltpu.sync_copy(acc_sh, out_hbm.at[pl.ds(pl.multiple_of(c * n, 8), n)])


def _segsum(edge_attr, receivers, zeros):
    e, h = edge_attr.shape
    n = zeros.shape[0]
    ew = e // NW
    nch = ew // CHUNK
    mesh = plsc.VectorSubcoreMesh(core_axis_name="c", subcore_axis_name="s")
    k = pl.kernel(
        functools.partial(_segsum_body, n, ew, nch),
        out_type=jax.ShapeDtypeStruct((NC * n, h), jnp.float32),
        mesh=mesh,
        scratch_types=[
            pltpu.VMEM((CHUNK,), jnp.int32),
            pltpu.VMEM((CHUNK,), jnp.int32),
            pltpu.VMEM((CHUNK, h), jnp.float32),
            pltpu.VMEM((CHUNK, h), jnp.float32),
            pltpu.VMEM_SHARED((n, h), jnp.float32),
            pltpu.SemaphoreType.DMA,
            pltpu.SemaphoreType.DMA,
            pltpu.SemaphoreType.DMA,
            pltpu.SemaphoreType.DMA,
        ],
    )
    return k(edge_attr, receivers, zeros)


def _gather_body(ew, nch,
                 xa_hbm, xb_hbm, snd_hbm, rcv_hbm, g_hbm,
                 idxs_all, idxr_all,
                 bufa0, bufb0, bufa1, bufb1, bufa2, bufb2, bufa3, bufb3,
                 gsem0, gsem1, gsem2, gsem3, wsem0, wsem1, wsem2, wsem3):
    c = lax.axis_index("c")
    s = lax.axis_index("s")
    wid = c * NS + s
    base_e = pl.multiple_of(wid * ew, CHUNK)

    bufs = [(bufa0, bufb0, gsem0, wsem0), (bufa1, bufb1, gsem1, wsem1),
            (bufa2, bufb2, gsem2, wsem2), (bufa3, bufb3, gsem3, wsem3)]

    pltpu.sync_copy(snd_hbm.at[pl.ds(base_e, ew)], idxs_all)
    pltpu.sync_copy(rcv_hbm.at[pl.ds(base_e, ew)], idxr_all)

    def islice(ref, i):
        return ref.at[pl.ds(pl.multiple_of(i * CHUNK, CHUNK), CHUNK)]

    def issue(i, sl):
        ba, bb, gsem, _ = bufs[sl]
        pltpu.async_copy(xa_hbm.at[islice(idxs_all, i)], ba, gsem)
        pltpu.async_copy(xb_hbm.at[islice(idxr_all, i)], bb, gsem)

    def wait_gather(sl):
        ba, bb, gsem, _ = bufs[sl]
        pltpu.make_async_copy(xa_hbm.at[pl.ds(0, CHUNK)], ba, gsem).wait()
        pltpu.make_async_copy(xb_hbm.at[pl.ds(0, CHUNK)], bb, gsem).wait()

    def add_wb(i, sl):
        ba, bb, _, wsem = bufs[sl]

        def row(j, carry2):
            for kk in range(8):
                plsc.addupdate(ba.at[j, pl.ds(kk * 16, 16)],
                               bb[j, pl.ds(kk * 16, 16)])
            return carry2

        lax.fori_loop(0, CHUNK, row, 0)
        b = pl.multiple_of(base_e + i * CHUNK, CHUNK)
        pltpu.async_copy(ba, g_hbm.at[pl.ds(b, CHUNK)], wsem)

    def wait_wb(sl):
        ba, _, _, wsem = bufs[sl]
        pltpu.make_async_copy(ba, g_hbm.at[pl.ds(0, CHUNK)], wsem).wait()

    issue(0, 0)
    issue(1, 1)

    def group(j, carry):
        for s4 in range(4):
            cc = 4 * j + s4

            @pl.when(cc + 2 < nch)
            def _():
                @pl.when(cc >= 2)
                def _():
                    wait_wb((s4 + 2) % 4)
                issue(cc + 2, (s4 + 2) % 4)

            wait_gather(s4)
            add_wb(cc, s4)
        return carry

    lax.fori_loop(0, nch // 4, group, 0)
    for s4 in range(nch % 4):
        cc = (nch // 4) * 4 + s4
        wait_gather(s4)
        add_wb(cc, s4)
    for s4 in range(4):
        wait_wb(s4)


def _gather_add(xa, xb, senders, receivers):
    n, h = xa.shape
    e = senders.shape[0]
    ew = e // NW
    nch = ew // CHUNK
    mesh = plsc.VectorSubcoreMesh(core_axis_name="c", subcore_axis_name="s")
    k = pl.kernel(
        functools.partial(_gather_body, ew, nch),
        out_type=jax.ShapeDtypeStruct((e, h), jnp.float32),
        mesh=mesh,
        scratch_types=(
            [pltpu.VMEM((ew,), jnp.int32)] * 2
            + [pltpu.VMEM((CHUNK, h), jnp.float32)] * 8
            + [pltpu.SemaphoreType.DMA] * 8
        ),
    )
    return k(xa, xb, senders, receivers)


def _layer_norm(hh, gamma, beta):
    mu = jnp.mean(hh, axis=-1, keepdims=True)
    var = jnp.mean((hh - mu) ** 2, axis=-1, keepdims=True)
    return (hh - mu) * lax.rsqrt(var + 1e-5) * gamma + beta


def _cell_body(n, x_ref, na_ref, aggp_ref,
               w1a_ref, w1b_ref, b1_ref, w2_ref, b2_ref, w3_ref, b3_ref,
               g_ref, bt_ref, ew1a_ref, ew1b_ref,
               xout_ref, xa_ref, xb_ref):
    f32 = jnp.float32
    agg = aggp_ref[:n, :] + aggp_ref[n:, :]
    nb = jnp.dot(na_ref[...], w1b_ref[...], preferred_element_type=f32) + b1_ref[...]

    def mlp(xin):
        hh = jax.nn.silu(
            jnp.dot(xin + agg, w1a_ref[...], preferred_element_type=f32) + nb)
        hh = jax.nn.silu(
            jnp.dot(hh, w2_ref[...], preferred_element_type=f32) + b2_ref[...])
        hh = jnp.dot(hh, w3_ref[...], preferred_element_type=f32) + b3_ref[...]
        return _layer_norm(hh, g_ref[...], bt_ref[...])

    x0 = x_ref[...]
    x2 = mlp(mlp(x0))
    xout_ref[...] = x0 + x2
    xa_ref[...] = jnp.dot(x2, ew1a_ref[...], preferred_element_type=f32)
    xb_ref[...] = jnp.dot(x2, ew1b_ref[...], preferred_element_type=f32)


def _cell(x, node_attr, aggp, cb_W1, cb_b1, cb_W2, cb_b2, cb_W3, cb_b3,
          cb_g, cb_bt, eb_W1):
    n, h = x.shape
    w1a, w1b = cb_W1[:h], cb_W1[h:]
    ew1a, ew1b = eb_W1[:h], eb_W1[h:2 * h]
    row = lambda v: v.reshape(1, h)
    out = pl.pallas_call(
        functools.partial(_cell_body, n),
        out_shape=[jax.ShapeDtypeStruct((n, h), jnp.float32)] * 3,
    )(x, node_attr, aggp, w1a, w1b, row(cb_b1), cb_W2, row(cb_b2),
      cb_W3, row(cb_b3), row(cb_g), row(cb_bt), ew1a, ew1b)
    return out


def _edge_body(g_ref, e_ref, w1c_ref, b1_ref, w2_ref, b2_ref, w3_ref, b3_ref,
               gm_ref, bt_ref, out_ref):
    f32 = jnp.float32
    e0 = e_ref[...]
    hh = jax.nn.silu(
        g_ref[...] + jnp.dot(e0, w1c_ref[...], preferred_element_type=f32)
        + b1_ref[...])
    hh = jax.nn.silu(
        jnp.dot(hh, w2_ref[...], preferred_element_type=f32) + b2_ref[...])
    hh = jnp.dot(hh, w3_ref[...], preferred_element_type=f32) + b3_ref[...]
    out_ref[...] = e0 + _layer_norm(hh, gm_ref[...], bt_ref[...])


def _edge(g, edge_attr, eb_W1, eb_b1, eb_W2, eb_b2, eb_W3, eb_b3, eb_g, eb_bt):
    e, h = edge_attr.shape
    r = 4000
    w1c = eb_W1[2 * h:]
    row = lambda v: v.reshape(1, h)
    blk = pl.BlockSpec((r, h), lambda i: (i, 0))
    wspec = pl.BlockSpec((h, h), lambda i: (0, 0))
    bspec = pl.BlockSpec((1, h), lambda i: (0, 0))
    return pl.pallas_call(
        _edge_body,
        grid=(e // r,),
        in_specs=[blk, blk, wspec, bspec, wspec, bspec, wspec, bspec,
                  bspec, bspec],
        out_specs=blk,
        out_shape=jax.ShapeDtypeStruct((e, h), jnp.float32),
    )(g, edge_attr, w1c, row(eb_b1), eb_W2, row(eb_b2), eb_W3, row(eb_b3),
      row(eb_g), row(eb_bt))


def kernel(x, edge_attr, node_attr, edge_index,
           cb_W1, cb_b1, cb_W2, cb_b2, cb_W3, cb_b3, cb_g, cb_bt,
           eb_W1, eb_b1, eb_W2, eb_b2, eb_W3, eb_b3, eb_g, eb_bt):
    senders = edge_index[0]
    receivers = edge_index[1]
    n, h = x.shape
    zeros = jnp.zeros((n, h), jnp.float32)
    aggp = _segsum(edge_attr, receivers, zeros)
    x_out, xa, xb = _cell(x, node_attr, aggp, cb_W1, cb_b1, cb_W2, cb_b2,
                          cb_W3, cb_b3, cb_g, cb_bt, eb_W1)
    g = _gather_add(xa, xb, senders, receivers)
    e_out = _edge(g, edge_attr, eb_W1, eb_b1, eb_W2, eb_b2, eb_W3, eb_b3,
                  eb_g, eb_bt)
    return (x_out, e_out)


# speedup vs baseline: 1.3431x; 1.0258x over previous
import functools

import jax
import jax.numpy as jnp
from jax import lax
from jax.experimental import pallas as pl
from jax.experimental.pallas import tpu as pltpu
from jax.experimental.pallas import tpu_sc as plsc

NC = 2
NS = 16
NW = NC * NS
CHUNK = 80


def _segsum_body(n, ew, nch, stripe,
                 edge_hbm, recv_hbm, zeros_hbm, out_hbm,
                 idx0, idx1, idx2, idx3, rows0, rows1, rows2, rows3, acc_sh,
                 lsem0, lsem1, lsem2, lsem3, ssem0, ssem1, ssem2, ssem3):
    c = lax.axis_index("c")
    s = lax.axis_index("s")
    wid = c * NS + s
    last = stripe * (NS - 1)
    rest = n - last

    @pl.when(s < NS - 1)
    def _():
        b = pl.multiple_of(s * stripe, 8)
        pltpu.sync_copy(zeros_hbm.at[pl.ds(b, stripe)],
                        acc_sh.at[pl.ds(b, stripe)])

    @pl.when(s == NS - 1)
    def _():
        pltpu.sync_copy(zeros_hbm.at[pl.ds(last, rest)],
                        acc_sh.at[pl.ds(last, rest)])

    plsc.subcore_barrier()
    base_e = pl.multiple_of(wid * ew, CHUNK)
    rows = [(idx0, rows0, lsem0, ssem0), (idx1, rows1, lsem1, ssem1),
            (idx2, rows2, lsem2, ssem2), (idx3, rows3, lsem3, ssem3)]

    def load(i, sl):
        iv, rv, lsem, _ = rows[sl]
        b = pl.multiple_of(base_e + i * CHUNK, CHUNK)
        pltpu.async_copy(recv_hbm.at[pl.ds(b, CHUNK)], iv, lsem)
        pltpu.async_copy(edge_hbm.at[pl.ds(b, CHUNK)], rv, lsem)

    def wait_load(sl):
        iv, rv, lsem, _ = rows[sl]
        pltpu.make_async_copy(recv_hbm.at[pl.ds(0, CHUNK)], iv, lsem).wait()
        pltpu.make_async_copy(edge_hbm.at[pl.ds(0, CHUNK)], rv, lsem).wait()

    def scatter(i, sl):
        iv, rv, _, ssem = rows[sl]
        pltpu.async_copy(rv, acc_sh.at[iv], ssem, add=True)

    def wait_scatter(sl):
        iv, rv, _, ssem = rows[sl]
        pltpu.make_async_copy(rv, acc_sh.at[iv], ssem).wait()

    load(0, 0)
    load(1, 1)

    def group(j, carry):
        for s4 in range(4):
            cc = 4 * j + s4

            @pl.when(cc + 2 < nch)
            def _():
                @pl.when(cc >= 2)
                def _():
                    wait_scatter((s4 + 2) % 4)
                load(cc + 2, (s4 + 2) % 4)

            wait_load(s4)
            scatter(cc, s4)
        return carry

    lax.fori_loop(0, nch // 4, group, 0)
    for s4 in range(nch % 4):
        cc = (nch // 4) * 4 + s4
        wait_load(s4)
        scatter(cc, s4)
    for s4 in range(4):
        wait_scatter(s4)
    plsc.subcore_barrier()

    base_o = pl.multiple_of(c * n, 8)

    @pl.when(s < NS - 1)
    def _():
        b = pl.multiple_of(s * stripe, 8)
        pltpu.sync_copy(acc_sh.at[pl.ds(b, stripe)],
                        out_hbm.at[pl.ds(base_o + b, stripe)])

    @pl.when(s == NS - 1)
    def _():
        pltpu.sync_copy(acc_sh.at[pl.ds(last, rest)],
                        out_hbm.at[pl.ds(base_o + last, rest)])


def _segsum(edge_attr, receivers, zeros):
    e, h = edge_attr.shape
    n = zeros.shape[0]
    ew = e // NW
    nch = ew // CHUNK
    stripe = (n // NS) // 8 * 8
    mesh = plsc.VectorSubcoreMesh(core_axis_name="c", subcore_axis_name="s")
    k = pl.kernel(
        functools.partial(_segsum_body, n, ew, nch, stripe),
        out_type=jax.ShapeDtypeStruct((NC * n, h), jnp.float32),
        mesh=mesh,
        scratch_types=(
            [pltpu.VMEM((CHUNK,), jnp.int32)] * 4
            + [pltpu.VMEM((CHUNK, h), jnp.float32)] * 4
            + [pltpu.VMEM_SHARED((n, h), jnp.float32)]
            + [pltpu.SemaphoreType.DMA] * 8
        ),
    )
    return k(edge_attr, receivers, zeros)


def _gather_body(ew, nch,
                 xa_hbm, xb_hbm, snd_hbm, rcv_hbm, g_hbm,
                 idxs_all, idxr_all,
                 bufa0, bufb0, bufa1, bufb1, bufa2, bufb2, bufa3, bufb3,
                 gsem0, gsem1, gsem2, gsem3, wsem0, wsem1, wsem2, wsem3):
    c = lax.axis_index("c")
    s = lax.axis_index("s")
    wid = c * NS + s
    base_e = pl.multiple_of(wid * ew, CHUNK)

    bufs = [(bufa0, bufb0, gsem0, wsem0), (bufa1, bufb1, gsem1, wsem1),
            (bufa2, bufb2, gsem2, wsem2), (bufa3, bufb3, gsem3, wsem3)]

    pltpu.sync_copy(snd_hbm.at[pl.ds(base_e, ew)], idxs_all)
    pltpu.sync_copy(rcv_hbm.at[pl.ds(base_e, ew)], idxr_all)

    def islice(ref, i):
        return ref.at[pl.ds(pl.multiple_of(i * CHUNK, CHUNK), CHUNK)]

    def issue(i, sl):
        ba, bb, gsem, _ = bufs[sl]
        pltpu.async_copy(xa_hbm.at[islice(idxs_all, i)], ba, gsem)
        pltpu.async_copy(xb_hbm.at[islice(idxr_all, i)], bb, gsem)

    def wait_gather(sl):
        ba, bb, gsem, _ = bufs[sl]
        pltpu.make_async_copy(xa_hbm.at[pl.ds(0, CHUNK)], ba, gsem).wait()
        pltpu.make_async_copy(xb_hbm.at[pl.ds(0, CHUNK)], bb, gsem).wait()

    def add_wb(i, sl):
        ba, bb, _, wsem = bufs[sl]

        def row(j, carry2):
            for kk in range(8):
                plsc.addupdate(ba.at[j, pl.ds(kk * 16, 16)],
                               bb[j, pl.ds(kk * 16, 16)])
            return carry2

        lax.fori_loop(0, CHUNK, row, 0)
        b = pl.multiple_of(base_e + i * CHUNK, CHUNK)
        pltpu.async_copy(ba, g_hbm.at[pl.ds(b, CHUNK)], wsem)

    def wait_wb(sl):
        ba, _, _, wsem = bufs[sl]
        pltpu.make_async_copy(ba, g_hbm.at[pl.ds(0, CHUNK)], wsem).wait()

    issue(0, 0)
    issue(1, 1)

    def group(j, carry):
        for s4 in range(4):
            cc = 4 * j + s4

            @pl.when(cc + 2 < nch)
            def _():
                @pl.when(cc >= 2)
                def _():
                    wait_wb((s4 + 2) % 4)
                issue(cc + 2, (s4 + 2) % 4)

            wait_gather(s4)
            add_wb(cc, s4)
        return carry

    lax.fori_loop(0, nch // 4, group, 0)
    for s4 in range(nch % 4):
        cc = (nch // 4) * 4 + s4
        wait_gather(s4)
        add_wb(cc, s4)
    for s4 in range(4):
        wait_wb(s4)


def _gather_add(xa, xb, senders, receivers):
    n, h = xa.shape
    e = senders.shape[0]
    ew = e // NW
    nch = ew // CHUNK
    mesh = plsc.VectorSubcoreMesh(core_axis_name="c", subcore_axis_name="s")
    k = pl.kernel(
        functools.partial(_gather_body, ew, nch),
        out_type=jax.ShapeDtypeStruct((e, h), jnp.float32),
        mesh=mesh,
        scratch_types=(
            [pltpu.VMEM((ew,), jnp.int32)] * 2
            + [pltpu.VMEM((CHUNK, h), jnp.float32)] * 8
            + [pltpu.SemaphoreType.DMA] * 8
        ),
    )
    return k(xa, xb, senders, receivers)


def _layer_norm(hh, gamma, beta):
    mu = jnp.mean(hh, axis=-1, keepdims=True)
    var = jnp.mean((hh - mu) ** 2, axis=-1, keepdims=True)
    return (hh - mu) * lax.rsqrt(var + 1e-5) * gamma + beta


def _cell_body(n, x_ref, na_ref, aggp_ref,
               w1a_ref, w1b_ref, b1_ref, w2_ref, b2_ref, w3_ref, b3_ref,
               g_ref, bt_ref, ew1a_ref, ew1b_ref,
               xout_ref, xa_ref, xb_ref):
    f32 = jnp.float32
    agg = aggp_ref[:n, :] + aggp_ref[n:, :]
    nb = jnp.dot(na_ref[...], w1b_ref[...], preferred_element_type=f32) + b1_ref[...]

    def mlp(xin):
        hh = jax.nn.silu(
            jnp.dot(xin + agg, w1a_ref[...], preferred_element_type=f32) + nb)
        hh = jax.nn.silu(
            jnp.dot(hh, w2_ref[...], preferred_element_type=f32) + b2_ref[...])
        hh = jnp.dot(hh, w3_ref[...], preferred_element_type=f32) + b3_ref[...]
        return _layer_norm(hh, g_ref[...], bt_ref[...])

    x0 = x_ref[...]
    x2 = mlp(mlp(x0))
    xout_ref[...] = x0 + x2
    xa_ref[...] = jnp.dot(x2, ew1a_ref[...], preferred_element_type=f32)
    xb_ref[...] = jnp.dot(x2, ew1b_ref[...], preferred_element_type=f32)


def _cell(x, node_attr, aggp, cb_W1, cb_b1, cb_W2, cb_b2, cb_W3, cb_b3,
          cb_g, cb_bt, eb_W1):
    n, h = x.shape
    w1a, w1b = cb_W1[:h], cb_W1[h:]
    ew1a, ew1b = eb_W1[:h], eb_W1[h:2 * h]
    row = lambda v: v.reshape(1, h)
    out = pl.pallas_call(
        functools.partial(_cell_body, n),
        out_shape=[jax.ShapeDtypeStruct((n, h), jnp.float32)] * 3,
    )(x, node_attr, aggp, w1a, w1b, row(cb_b1), cb_W2, row(cb_b2),
      cb_W3, row(cb_b3), row(cb_g), row(cb_bt), ew1a, ew1b)
    return out


def _edge_body(g_ref, e_ref, w1c_ref, b1_ref, w2_ref, b2_ref, w3_ref, b3_ref,
               gm_ref, bt_ref, out_ref):
    f32 = jnp.float32
    e0 = e_ref[...]
    hh = jax.nn.silu(
        g_ref[...] + jnp.dot(e0, w1c_ref[...], preferred_element_type=f32)
        + b1_ref[...])
    hh = jax.nn.silu(
        jnp.dot(hh, w2_ref[...], preferred_element_type=f32) + b2_ref[...])
    hh = jnp.dot(hh, w3_ref[...], preferred_element_type=f32) + b3_ref[...]
    out_ref[...] = e0 + _layer_norm(hh, gm_ref[...], bt_ref[...])


def _edge(g, edge_attr, eb_W1, eb_b1, eb_W2, eb_b2, eb_W3, eb_b3, eb_g, eb_bt):
    e, h = edge_attr.shape
    r = 4000
    w1c = eb_W1[2 * h:]
    row = lambda v: v.reshape(1, h)
    blk = pl.BlockSpec((r, h), lambda i: (i, 0))
    wspec = pl.BlockSpec((h, h), lambda i: (0, 0))
    bspec = pl.BlockSpec((1, h), lambda i: (0, 0))
    return pl.pallas_call(
        _edge_body,
        grid=(e // r,),
        in_specs=[blk, blk, wspec, bspec, wspec, bspec, wspec, bspec,
                  bspec, bspec],
        out_specs=blk,
        out_shape=jax.ShapeDtypeStruct((e, h), jnp.float32),
    )(g, edge_attr, w1c, row(eb_b1), eb_W2, row(eb_b2), eb_W3, row(eb_b3),
      row(eb_g), row(eb_bt))


def kernel(x, edge_attr, node_attr, edge_index,
           cb_W1, cb_b1, cb_W2, cb_b2, cb_W3, cb_b3, cb_g, cb_bt,
           eb_W1, eb_b1, eb_W2, eb_b2, eb_W3, eb_b3, eb_g, eb_bt):
    senders = edge_index[0]
    receivers = edge_index[1]
    n, h = x.shape
    zeros = jnp.zeros((n, h), jnp.float32)
    aggp = _segsum(edge_attr, receivers, zeros)
    x_out, xa, xb = _cell(x, node_attr, aggp, cb_W1, cb_b1, cb_W2, cb_b2,
                          cb_W3, cb_b3, cb_g, cb_bt, eb_W1)
    g = _gather_add(xa, xb, senders, receivers)
    e_out = _edge(g, edge_attr, eb_W1, eb_b1, eb_W2, eb_b2, eb_W3, eb_b3,
                  eb_g, eb_bt)
    return (x_out, e_out)


# speedup vs baseline: 1.3555x; 1.0092x over previous
import functools

import jax
import jax.numpy as jnp
from jax import lax
from jax.experimental import pallas as pl
from jax.experimental.pallas import tpu as pltpu
from jax.experimental.pallas import tpu_sc as plsc

NC = 2
NS = 16
NW = NC * NS
CHUNK = 80


def _segsum_body(n, ew, nch, stripe,
                 edge_hbm, recv_hbm, zeros_hbm, out_hbm,
                 idx0, idx1, idx2, idx3, rows0, rows1, rows2, rows3, acc_sh,
                 lsem0, lsem1, lsem2, lsem3, ssem0, ssem1, ssem2, ssem3):
    c = lax.axis_index("c")
    s = lax.axis_index("s")
    wid = c * NS + s
    last = stripe * (NS - 1)
    rest = n - last

    @pl.when(s < NS - 1)
    def _():
        b = pl.multiple_of(s * stripe, 8)
        pltpu.sync_copy(zeros_hbm.at[pl.ds(b, stripe)],
                        acc_sh.at[pl.ds(b, stripe)])

    @pl.when(s == NS - 1)
    def _():
        pltpu.sync_copy(zeros_hbm.at[pl.ds(last, rest)],
                        acc_sh.at[pl.ds(last, rest)])

    plsc.subcore_barrier()
    base_e = pl.multiple_of(wid * ew, CHUNK)
    rows = [(idx0, rows0, lsem0, ssem0), (idx1, rows1, lsem1, ssem1),
            (idx2, rows2, lsem2, ssem2), (idx3, rows3, lsem3, ssem3)]

    def load(i, sl):
        iv, rv, lsem, _ = rows[sl]
        b = pl.multiple_of(base_e + i * CHUNK, CHUNK)
        pltpu.async_copy(recv_hbm.at[pl.ds(b, CHUNK)], iv, lsem)
        pltpu.async_copy(edge_hbm.at[pl.ds(b, CHUNK)], rv, lsem)

    def wait_load(sl):
        iv, rv, lsem, _ = rows[sl]
        pltpu.make_async_copy(recv_hbm.at[pl.ds(0, CHUNK)], iv, lsem).wait()
        pltpu.make_async_copy(edge_hbm.at[pl.ds(0, CHUNK)], rv, lsem).wait()

    def scatter(i, sl):
        iv, rv, _, ssem = rows[sl]
        pltpu.async_copy(rv, acc_sh.at[iv], ssem, add=True)

    def wait_scatter(sl):
        iv, rv, _, ssem = rows[sl]
        pltpu.make_async_copy(rv, acc_sh.at[iv], ssem).wait()

    load(0, 0)
    load(1, 1)

    def group(j, carry):
        for s4 in range(4):
            cc = 4 * j + s4

            @pl.when(cc + 2 < nch)
            def _():
                @pl.when(cc >= 2)
                def _():
                    wait_scatter((s4 + 2) % 4)
                load(cc + 2, (s4 + 2) % 4)

            wait_load(s4)
            scatter(cc, s4)
        return carry

    lax.fori_loop(0, nch // 4, group, 0)
    for s4 in range(nch % 4):
        cc = (nch // 4) * 4 + s4
        wait_load(s4)
        scatter(cc, s4)
    for s4 in range(4):
        wait_scatter(s4)
    plsc.subcore_barrier()

    base_o = pl.multiple_of(c * n, 8)

    @pl.when(s < NS - 1)
    def _():
        b = pl.multiple_of(s * stripe, 8)
        pltpu.sync_copy(acc_sh.at[pl.ds(b, stripe)],
                        out_hbm.at[pl.ds(base_o + b, stripe)])

    @pl.when(s == NS - 1)
    def _():
        pltpu.sync_copy(acc_sh.at[pl.ds(last, rest)],
                        out_hbm.at[pl.ds(base_o + last, rest)])


def _segsum(edge_attr, receivers, zeros):
    e, h = edge_attr.shape
    n = zeros.shape[0]
    ew = e // NW
    nch = ew // CHUNK
    stripe = (n // NS) // 8 * 8
    mesh = plsc.VectorSubcoreMesh(core_axis_name="c", subcore_axis_name="s")
    k = pl.kernel(
        functools.partial(_segsum_body, n, ew, nch, stripe),
        out_type=jax.ShapeDtypeStruct((NC * n, h), jnp.float32),
        mesh=mesh,
        scratch_types=(
            [pltpu.VMEM((CHUNK,), jnp.int32)] * 4
            + [pltpu.VMEM((CHUNK, h), jnp.float32)] * 4
            + [pltpu.VMEM_SHARED((n, h), jnp.float32)]
            + [pltpu.SemaphoreType.DMA] * 8
        ),
    )
    return k(edge_attr, receivers, zeros)


def _gather_body(ew, nch,
                 xa_hbm, xb_hbm, snd_hbm, rcv_hbm, g_hbm,
                 idxs_all, idxr_all,
                 bufa0, bufb0, bufa1, bufb1, bufa2, bufb2, bufa3, bufb3,
                 gsem0, gsem1, gsem2, gsem3, wsem0, wsem1, wsem2, wsem3):
    c = lax.axis_index("c")
    s = lax.axis_index("s")
    wid = c * NS + s
    base_e = pl.multiple_of(wid * ew, CHUNK)

    bufs = [(bufa0, bufb0, gsem0, wsem0), (bufa1, bufb1, gsem1, wsem1),
            (bufa2, bufb2, gsem2, wsem2), (bufa3, bufb3, gsem3, wsem3)]

    pltpu.sync_copy(snd_hbm.at[pl.ds(base_e, ew)], idxs_all)
    pltpu.sync_copy(rcv_hbm.at[pl.ds(base_e, ew)], idxr_all)

    def islice(ref, i):
        return ref.at[pl.ds(pl.multiple_of(i * CHUNK, CHUNK), CHUNK)]

    def issue(i, sl):
        ba, bb, gsem, _ = bufs[sl]
        pltpu.async_copy(xa_hbm.at[islice(idxs_all, i)], ba, gsem)
        pltpu.async_copy(xb_hbm.at[islice(idxr_all, i)], bb, gsem)

    def wait_gather(sl):
        ba, bb, gsem, _ = bufs[sl]
        pltpu.make_async_copy(xa_hbm.at[pl.ds(0, CHUNK)], ba, gsem).wait()
        pltpu.make_async_copy(xb_hbm.at[pl.ds(0, CHUNK)], bb, gsem).wait()

    def add_wb(i, sl):
        ba, bb, _, wsem = bufs[sl]

        def row(j, carry2):
            for kk in range(8):
                plsc.addupdate(ba.at[j, pl.ds(kk * 16, 16)],
                               bb[j, pl.ds(kk * 16, 16)])
            return carry2

        lax.fori_loop(0, CHUNK, row, 0)
        b = pl.multiple_of(base_e + i * CHUNK, CHUNK)
        pltpu.async_copy(ba, g_hbm.at[pl.ds(b, CHUNK)], wsem)

    def wait_wb(sl):
        ba, _, _, wsem = bufs[sl]
        pltpu.make_async_copy(ba, g_hbm.at[pl.ds(0, CHUNK)], wsem).wait()

    issue(0, 0)
    issue(1, 1)

    def group(j, carry):
        for s4 in range(4):
            cc = 4 * j + s4

            @pl.when(cc + 2 < nch)
            def _():
                @pl.when(cc >= 2)
                def _():
                    wait_wb((s4 + 2) % 4)
                issue(cc + 2, (s4 + 2) % 4)

            wait_gather(s4)
            add_wb(cc, s4)
        return carry

    lax.fori_loop(0, nch // 4, group, 0)
    for s4 in range(nch % 4):
        cc = (nch // 4) * 4 + s4
        wait_gather(s4)
        add_wb(cc, s4)
    for s4 in range(4):
        wait_wb(s4)


def _gather_add(xa, xb, senders, receivers):
    n, h = xa.shape
    e = senders.shape[0]
    ew = e // NW
    nch = ew // CHUNK
    mesh = plsc.VectorSubcoreMesh(core_axis_name="c", subcore_axis_name="s")
    k = pl.kernel(
        functools.partial(_gather_body, ew, nch),
        out_type=jax.ShapeDtypeStruct((e, h), jnp.float32),
        mesh=mesh,
        scratch_types=(
            [pltpu.VMEM((ew,), jnp.int32)] * 2
            + [pltpu.VMEM((CHUNK, h), jnp.float32)] * 8
            + [pltpu.SemaphoreType.DMA] * 8
        ),
    )
    return k(xa, xb, senders, receivers)


def _layer_norm(hh, gamma, beta):
    mu = jnp.mean(hh, axis=-1, keepdims=True)
    var = jnp.mean((hh - mu) ** 2, axis=-1, keepdims=True)
    return (hh - mu) * lax.rsqrt(var + 1e-5) * gamma + beta


def _cell_body(x_ref, na_ref, agg0_ref, agg1_ref,
               w1a_ref, w1b_ref, b1_ref, w2_ref, b2_ref, w3_ref, b3_ref,
               g_ref, bt_ref, ew1a_ref, ew1b_ref,
               xout_ref, xa_ref, xb_ref):
    f32 = jnp.float32
    agg = agg0_ref[...] + agg1_ref[...]
    nb = jnp.dot(na_ref[...], w1b_ref[...], preferred_element_type=f32) + b1_ref[...]

    def mlp(xin):
        hh = jax.nn.silu(
            jnp.dot(xin + agg, w1a_ref[...], preferred_element_type=f32) + nb)
        hh = jax.nn.silu(
            jnp.dot(hh, w2_ref[...], preferred_element_type=f32) + b2_ref[...])
        hh = jnp.dot(hh, w3_ref[...], preferred_element_type=f32) + b3_ref[...]
        return _layer_norm(hh, g_ref[...], bt_ref[...])

    x0 = x_ref[...]
    x2 = mlp(mlp(x0))
    xout_ref[...] = x0 + x2
    xa_ref[...] = jnp.dot(x2, ew1a_ref[...], preferred_element_type=f32)
    xb_ref[...] = jnp.dot(x2, ew1b_ref[...], preferred_element_type=f32)


def _cell(x, node_attr, aggp, cb_W1, cb_b1, cb_W2, cb_b2, cb_W3, cb_b3,
          cb_g, cb_bt, eb_W1):
    n, h = x.shape
    hn = node_attr.shape[1]
    w1a, w1b = cb_W1[:h], cb_W1[h:]
    ew1a, ew1b = eb_W1[:h], eb_W1[h:2 * h]
    row = lambda v: v.reshape(1, h)
    r = 2000
    blk = pl.BlockSpec((r, h), lambda i: (i, 0))
    nblk = pl.BlockSpec((r, hn), lambda i: (i, 0))
    a1blk = pl.BlockSpec((r, h), lambda i: (i + n // r, 0))
    wspec = lambda rows: pl.BlockSpec((rows, h), lambda i: (0, 0))
    out = pl.pallas_call(
        _cell_body,
        grid=(n // r,),
        in_specs=[blk, nblk, blk, a1blk, wspec(h), wspec(hn), wspec(1),
                  wspec(h), wspec(1), wspec(h), wspec(1), wspec(1), wspec(1),
                  wspec(h), wspec(h)],
        out_specs=[blk] * 3,
        out_shape=[jax.ShapeDtypeStruct((n, h), jnp.float32)] * 3,
    )(x, node_attr, aggp, aggp, w1a, w1b, row(cb_b1), cb_W2, row(cb_b2),
      cb_W3, row(cb_b3), row(cb_g), row(cb_bt), ew1a, ew1b)
    return out


def _edge_body(g_ref, e_ref, w1c_ref, b1_ref, w2_ref, b2_ref, w3_ref, b3_ref,
               gm_ref, bt_ref, out_ref):
    f32 = jnp.float32
    e0 = e_ref[...]
    hh = jax.nn.silu(
        g_ref[...] + jnp.dot(e0, w1c_ref[...], preferred_element_type=f32)
        + b1_ref[...])
    hh = jax.nn.silu(
        jnp.dot(hh, w2_ref[...], preferred_element_type=f32) + b2_ref[...])
    hh = jnp.dot(hh, w3_ref[...], preferred_element_type=f32) + b3_ref[...]
    out_ref[...] = e0 + _layer_norm(hh, gm_ref[...], bt_ref[...])


def _edge(g, edge_attr, eb_W1, eb_b1, eb_W2, eb_b2, eb_W3, eb_b3, eb_g, eb_bt):
    e, h = edge_attr.shape
    r = 4000
    w1c = eb_W1[2 * h:]
    row = lambda v: v.reshape(1, h)
    blk = pl.BlockSpec((r, h), lambda i: (i, 0))
    wspec = pl.BlockSpec((h, h), lambda i: (0, 0))
    bspec = pl.BlockSpec((1, h), lambda i: (0, 0))
    return pl.pallas_call(
        _edge_body,
        grid=(e // r,),
        in_specs=[blk, blk, wspec, bspec, wspec, bspec, wspec, bspec,
                  bspec, bspec],
        out_specs=blk,
        out_shape=jax.ShapeDtypeStruct((e, h), jnp.float32),
    )(g, edge_attr, w1c, row(eb_b1), eb_W2, row(eb_b2), eb_W3, row(eb_b3),
      row(eb_g), row(eb_bt))


def kernel(x, edge_attr, node_attr, edge_index,
           cb_W1, cb_b1, cb_W2, cb_b2, cb_W3, cb_b3, cb_g, cb_bt,
           eb_W1, eb_b1, eb_W2, eb_b2, eb_W3, eb_b3, eb_g, eb_bt):
    senders = edge_index[0]
    receivers = edge_index[1]
    n, h = x.shape
    zeros = jnp.zeros((n, h), jnp.float32)
    aggp = _segsum(edge_attr, receivers, zeros)
    x_out, xa, xb = _cell(x, node_attr, aggp, cb_W1, cb_b1, cb_W2, cb_b2,
                          cb_W3, cb_b3, cb_g, cb_bt, eb_W1)
    g = _gather_add(xa, xb, senders, receivers)
    e_out = _edge(g, edge_attr, eb_W1, eb_b1, eb_W2, eb_b2, eb_W3, eb_b3,
                  eb_g, eb_bt)
    return (x_out, e_out)


# speedup vs baseline: 1.4154x; 1.0442x over previous
import functools

import jax
import jax.numpy as jnp
from jax import lax
from jax.experimental import pallas as pl
from jax.experimental.pallas import tpu as pltpu
from jax.experimental.pallas import tpu_sc as plsc

NC = 2
NS = 16
NW = NC * NS
CHUNK = 80


def _segsum_body(n, ew, nch, stripe,
                 edge_hbm, recv_hbm, zeros_hbm, out_hbm,
                 idx0, idx1, idx2, idx3, rows0, rows1, rows2, rows3, acc_sh,
                 lsem0, lsem1, lsem2, lsem3, ssem0, ssem1, ssem2, ssem3):
    c = lax.axis_index("c")
    s = lax.axis_index("s")
    wid = c * NS + s
    last = stripe * (NS - 1)
    rest = n - last

    @pl.when(s < NS - 1)
    def _():
        b = pl.multiple_of(s * stripe, 8)
        pltpu.sync_copy(zeros_hbm.at[pl.ds(b, stripe)],
                        acc_sh.at[pl.ds(b, stripe)])

    @pl.when(s == NS - 1)
    def _():
        pltpu.sync_copy(zeros_hbm.at[pl.ds(last, rest)],
                        acc_sh.at[pl.ds(last, rest)])

    plsc.subcore_barrier()
    base_e = pl.multiple_of(wid * ew, CHUNK)
    rows = [(idx0, rows0, lsem0, ssem0), (idx1, rows1, lsem1, ssem1),
            (idx2, rows2, lsem2, ssem2), (idx3, rows3, lsem3, ssem3)]

    def load(i, sl):
        iv, rv, lsem, _ = rows[sl]
        b = pl.multiple_of(base_e + i * CHUNK, CHUNK)
        pltpu.async_copy(recv_hbm.at[pl.ds(b, CHUNK)], iv, lsem)
        pltpu.async_copy(edge_hbm.at[pl.ds(b, CHUNK)], rv, lsem)

    def wait_load(sl):
        iv, rv, lsem, _ = rows[sl]
        pltpu.make_async_copy(recv_hbm.at[pl.ds(0, CHUNK)], iv, lsem).wait()
        pltpu.make_async_copy(edge_hbm.at[pl.ds(0, CHUNK)], rv, lsem).wait()

    def scatter(i, sl):
        iv, rv, _, ssem = rows[sl]
        pltpu.async_copy(rv, acc_sh.at[iv], ssem, add=True)

    def wait_scatter(sl):
        iv, rv, _, ssem = rows[sl]
        pltpu.make_async_copy(rv, acc_sh.at[iv], ssem).wait()

    load(0, 0)
    load(1, 1)

    def group(j, carry):
        for s4 in range(4):
            cc = 4 * j + s4

            @pl.when(cc + 2 < nch)
            def _():
                @pl.when(cc >= 2)
                def _():
                    wait_scatter((s4 + 2) % 4)
                load(cc + 2, (s4 + 2) % 4)

            wait_load(s4)
            scatter(cc, s4)
        return carry

    lax.fori_loop(0, nch // 4, group, 0)
    for s4 in range(nch % 4):
        cc = (nch // 4) * 4 + s4
        wait_load(s4)
        scatter(cc, s4)
    for s4 in range(4):
        wait_scatter(s4)
    plsc.subcore_barrier()

    base_o = pl.multiple_of(c * n, 8)

    @pl.when(s < NS - 1)
    def _():
        b = pl.multiple_of(s * stripe, 8)
        pltpu.sync_copy(acc_sh.at[pl.ds(b, stripe)],
                        out_hbm.at[pl.ds(base_o + b, stripe)])

    @pl.when(s == NS - 1)
    def _():
        pltpu.sync_copy(acc_sh.at[pl.ds(last, rest)],
                        out_hbm.at[pl.ds(base_o + last, rest)])


def _segsum(edge_attr, receivers, zeros):
    e, h = edge_attr.shape
    n = zeros.shape[0]
    ew = e // NW
    nch = ew // CHUNK
    stripe = (n // NS) // 8 * 8
    mesh = plsc.VectorSubcoreMesh(core_axis_name="c", subcore_axis_name="s")
    k = pl.kernel(
        functools.partial(_segsum_body, n, ew, nch, stripe),
        out_type=jax.ShapeDtypeStruct((NC * n, h), jnp.float32),
        mesh=mesh,
        scratch_types=(
            [pltpu.VMEM((CHUNK,), jnp.int32)] * 4
            + [pltpu.VMEM((CHUNK, h), jnp.float32)] * 4
            + [pltpu.VMEM_SHARED((n, h), jnp.float32)]
            + [pltpu.SemaphoreType.DMA] * 8
        ),
    )
    return k(edge_attr, receivers, zeros)


def _gather_body(ew, nch,
                 xa_hbm, xb_hbm, snd_hbm, rcv_hbm, g_hbm,
                 idxs_all, idxr_all,
                 bufa0, bufb0, bufa1, bufb1, bufa2, bufb2, bufa3, bufb3,
                 gsem0, gsem1, gsem2, gsem3, wsem0, wsem1, wsem2, wsem3):
    c = lax.axis_index("c")
    s = lax.axis_index("s")
    wid = c * NS + s
    base_e = pl.multiple_of(wid * ew, CHUNK)

    bufs = [(bufa0, bufb0, gsem0, wsem0), (bufa1, bufb1, gsem1, wsem1),
            (bufa2, bufb2, gsem2, wsem2), (bufa3, bufb3, gsem3, wsem3)]

    pltpu.sync_copy(snd_hbm.at[pl.ds(base_e, ew)], idxs_all)
    pltpu.sync_copy(rcv_hbm.at[pl.ds(base_e, ew)], idxr_all)

    def islice(ref, i):
        return ref.at[pl.ds(pl.multiple_of(i * CHUNK, CHUNK), CHUNK)]

    def issue(i, sl):
        ba, bb, gsem, _ = bufs[sl]
        pltpu.async_copy(xa_hbm.at[islice(idxs_all, i)], ba, gsem)
        pltpu.async_copy(xb_hbm.at[islice(idxr_all, i)], bb, gsem)

    def wait_gather(sl):
        ba, bb, gsem, _ = bufs[sl]
        pltpu.make_async_copy(xa_hbm.at[pl.ds(0, CHUNK)], ba, gsem).wait()
        pltpu.make_async_copy(xb_hbm.at[pl.ds(0, CHUNK)], bb, gsem).wait()

    def add_wb(i, sl):
        ba, bb, _, wsem = bufs[sl]

        def row(j, carry2):
            for kk in range(8):
                plsc.addupdate(ba.at[j, pl.ds(kk * 16, 16)],
                               bb[j, pl.ds(kk * 16, 16)])
            return carry2

        lax.fori_loop(0, CHUNK, row, 0)
        b = pl.multiple_of(base_e + i * CHUNK, CHUNK)
        pltpu.async_copy(ba, g_hbm.at[pl.ds(b, CHUNK)], wsem)

    def wait_wb(sl):
        ba, _, _, wsem = bufs[sl]
        pltpu.make_async_copy(ba, g_hbm.at[pl.ds(0, CHUNK)], wsem).wait()

    issue(0, 0)
    issue(1, 1)

    def group(j, carry):
        for s4 in range(4):
            cc = 4 * j + s4

            @pl.when(cc + 2 < nch)
            def _():
                @pl.when(cc >= 2)
                def _():
                    wait_wb((s4 + 2) % 4)
                issue(cc + 2, (s4 + 2) % 4)

            wait_gather(s4)
            add_wb(cc, s4)
        return carry

    lax.fori_loop(0, nch // 4, group, 0)
    for s4 in range(nch % 4):
        cc = (nch // 4) * 4 + s4
        wait_gather(s4)
        add_wb(cc, s4)
    for s4 in range(4):
        wait_wb(s4)


def _gather_add(xa, xb, senders, receivers):
    n, h = xa.shape
    e = senders.shape[0]
    ew = e // NW
    nch = ew // CHUNK
    mesh = plsc.VectorSubcoreMesh(core_axis_name="c", subcore_axis_name="s")
    k = pl.kernel(
        functools.partial(_gather_body, ew, nch),
        out_type=jax.ShapeDtypeStruct((e, h), jnp.float32),
        mesh=mesh,
        scratch_types=(
            [pltpu.VMEM((ew,), jnp.int32)] * 2
            + [pltpu.VMEM((CHUNK, h), jnp.float32)] * 8
            + [pltpu.SemaphoreType.DMA] * 8
        ),
    )
    return k(xa, xb, senders, receivers)


def _layer_norm(hh, gamma, beta):
    mu = jnp.mean(hh, axis=-1, keepdims=True)
    var = jnp.mean((hh - mu) ** 2, axis=-1, keepdims=True)
    return (hh - mu) * lax.rsqrt(var + 1e-5) * gamma + beta


def _cell_body(x_ref, na_ref, agg0_ref, agg1_ref,
               w1a_ref, w1b_ref, b1_ref, w2_ref, b2_ref, w3_ref, b3_ref,
               g_ref, bt_ref, ew1a_ref, ew1b_ref,
               xout_ref, xa_ref, xb_ref):
    f32 = jnp.float32
    agg = agg0_ref[...] + agg1_ref[...]
    nb = jnp.dot(na_ref[...], w1b_ref[...], preferred_element_type=f32) + b1_ref[...]

    def mlp(xin):
        hh = jax.nn.silu(
            jnp.dot(xin + agg, w1a_ref[...], preferred_element_type=f32) + nb)
        hh = jax.nn.silu(
            jnp.dot(hh, w2_ref[...], preferred_element_type=f32) + b2_ref[...])
        hh = jnp.dot(hh, w3_ref[...], preferred_element_type=f32) + b3_ref[...]
        return _layer_norm(hh, g_ref[...], bt_ref[...])

    x0 = x_ref[...]
    x2 = mlp(mlp(x0))
    xout_ref[...] = x0 + x2
    xa_ref[...] = jnp.dot(x2, ew1a_ref[...], preferred_element_type=f32)
    xb_ref[...] = jnp.dot(x2, ew1b_ref[...], preferred_element_type=f32)


def _cell(x, node_attr, aggp, cb_W1, cb_b1, cb_W2, cb_b2, cb_W3, cb_b3,
          cb_g, cb_bt, eb_W1):
    n, h = x.shape
    hn = node_attr.shape[1]
    w1a, w1b = cb_W1[:h], cb_W1[h:]
    ew1a, ew1b = eb_W1[:h], eb_W1[h:2 * h]
    row = lambda v: v.reshape(1, h)
    r = 2000
    blk = pl.BlockSpec((r, h), lambda i: (i, 0))
    nblk = pl.BlockSpec((r, hn), lambda i: (i, 0))
    a1blk = pl.BlockSpec((r, h), lambda i: (i + n // r, 0))
    wspec = lambda rows: pl.BlockSpec((rows, h), lambda i: (0, 0))
    out = pl.pallas_call(
        _cell_body,
        grid=(n // r,),
        in_specs=[blk, nblk, blk, a1blk, wspec(h), wspec(hn), wspec(1),
                  wspec(h), wspec(1), wspec(h), wspec(1), wspec(1), wspec(1),
                  wspec(h), wspec(h)],
        out_specs=[blk] * 3,
        out_shape=[jax.ShapeDtypeStruct((n, h), jnp.float32)] * 3,
    )(x, node_attr, aggp, aggp, w1a, w1b, row(cb_b1), cb_W2, row(cb_b2),
      cb_W3, row(cb_b3), row(cb_g), row(cb_bt), ew1a, ew1b)
    return out


def _edge_body(g_ref, e_ref, w1c_ref, b1_ref, w2_ref, b2_ref, w3_ref, b3_ref,
               gm_ref, bt_ref, out_ref):
    f32 = jnp.float32
    e0 = e_ref[...]
    hh = jax.nn.silu(
        g_ref[...] + jnp.dot(e0, w1c_ref[...], preferred_element_type=f32)
        + b1_ref[...])
    hh = jax.nn.silu(
        jnp.dot(hh, w2_ref[...], preferred_element_type=f32) + b2_ref[...])
    hh = jnp.dot(hh, w3_ref[...], preferred_element_type=f32) + b3_ref[...]
    out_ref[...] = e0 + _layer_norm(hh, gm_ref[...], bt_ref[...])


def _edge(g, edge_attr, eb_W1, eb_b1, eb_W2, eb_b2, eb_W3, eb_b3, eb_g, eb_bt):
    e, h = edge_attr.shape
    r = 8000
    w1c = eb_W1[2 * h:]
    row = lambda v: v.reshape(1, h)
    blk = pl.BlockSpec((r, h), lambda i: (i, 0))
    wspec = pl.BlockSpec((h, h), lambda i: (0, 0))
    bspec = pl.BlockSpec((1, h), lambda i: (0, 0))
    return pl.pallas_call(
        _edge_body,
        grid=(e // r,),
        in_specs=[blk, blk, wspec, bspec, wspec, bspec, wspec, bspec,
                  bspec, bspec],
        out_specs=blk,
        out_shape=jax.ShapeDtypeStruct((e, h), jnp.float32),
    )(g, edge_attr, w1c, row(eb_b1), eb_W2, row(eb_b2), eb_W3, row(eb_b3),
      row(eb_g), row(eb_bt))


def kernel(x, edge_attr, node_attr, edge_index,
           cb_W1, cb_b1, cb_W2, cb_b2, cb_W3, cb_b3, cb_g, cb_bt,
           eb_W1, eb_b1, eb_W2, eb_b2, eb_W3, eb_b3, eb_g, eb_bt):
    senders = edge_index[0]
    receivers = edge_index[1]
    n, h = x.shape
    zeros = jnp.zeros((n, h), jnp.float32)
    aggp = _segsum(edge_attr, receivers, zeros)
    x_out, xa, xb = _cell(x, node_attr, aggp, cb_W1, cb_b1, cb_W2, cb_b2,
                          cb_W3, cb_b3, cb_g, cb_bt, eb_W1)
    g = _gather_add(xa, xb, senders, receivers)
    e_out = _edge(g, edge_attr, eb_W1, eb_b1, eb_W2, eb_b2, eb_W3, eb_b3,
                  eb_g, eb_bt)
    return (x_out, e_out)


# speedup vs baseline: 1.4177x; 1.0016x over previous
import functools

import jax
import jax.numpy as jnp
from jax import lax
from jax.experimental import pallas as pl
from jax.experimental.pallas import tpu as pltpu
from jax.experimental.pallas import tpu_sc as plsc

NC = 2
NS = 16
NW = NC * NS
CHUNK = 80


def _segsum_body(n, ew, nch, stripe,
                 edge_hbm, recv_hbm, zeros_hbm, out_hbm,
                 idx0, idx1, idx2, idx3, rows0, rows1, rows2, rows3, acc_sh,
                 lsem0, lsem1, lsem2, lsem3, ssem0, ssem1, ssem2, ssem3):
    c = lax.axis_index("c")
    s = lax.axis_index("s")
    wid = c * NS + s
    last = stripe * (NS - 1)
    rest = n - last

    @pl.when(s < NS - 1)
    def _():
        b = pl.multiple_of(s * stripe, 8)
        pltpu.sync_copy(zeros_hbm.at[pl.ds(b, stripe)],
                        acc_sh.at[pl.ds(b, stripe)])

    @pl.when(s == NS - 1)
    def _():
        pltpu.sync_copy(zeros_hbm.at[pl.ds(last, rest)],
                        acc_sh.at[pl.ds(last, rest)])

    plsc.subcore_barrier()
    base_e = pl.multiple_of(wid * ew, CHUNK)
    rows = [(idx0, rows0, lsem0, ssem0), (idx1, rows1, lsem1, ssem1),
            (idx2, rows2, lsem2, ssem2), (idx3, rows3, lsem3, ssem3)]

    def load(i, sl):
        iv, rv, lsem, _ = rows[sl]
        b = pl.multiple_of(base_e + i * CHUNK, CHUNK)
        pltpu.async_copy(recv_hbm.at[pl.ds(b, CHUNK)], iv, lsem)
        pltpu.async_copy(edge_hbm.at[pl.ds(b, CHUNK)], rv, lsem)

    def wait_load(sl):
        iv, rv, lsem, _ = rows[sl]
        pltpu.make_async_copy(recv_hbm.at[pl.ds(0, CHUNK)], iv, lsem).wait()
        pltpu.make_async_copy(edge_hbm.at[pl.ds(0, CHUNK)], rv, lsem).wait()

    def scatter(i, sl):
        iv, rv, _, ssem = rows[sl]
        pltpu.async_copy(rv, acc_sh.at[iv], ssem, add=True)

    def wait_scatter(sl):
        iv, rv, _, ssem = rows[sl]
        pltpu.make_async_copy(rv, acc_sh.at[iv], ssem).wait()

    load(0, 0)
    load(1, 1)

    def group(j, carry):
        for s4 in range(4):
            cc = 4 * j + s4

            @pl.when(cc + 2 < nch)
            def _():
                @pl.when(cc >= 2)
                def _():
                    wait_scatter((s4 + 2) % 4)
                load(cc + 2, (s4 + 2) % 4)

            wait_load(s4)
            scatter(cc, s4)
        return carry

    lax.fori_loop(0, nch // 4, group, 0)
    for s4 in range(nch % 4):
        cc = (nch // 4) * 4 + s4
        wait_load(s4)
        scatter(cc, s4)
    for s4 in range(4):
        wait_scatter(s4)
    plsc.subcore_barrier()

    base_o = pl.multiple_of(c * n, 8)

    @pl.when(s < NS - 1)
    def _():
        b = pl.multiple_of(s * stripe, 8)
        pltpu.sync_copy(acc_sh.at[pl.ds(b, stripe)],
                        out_hbm.at[pl.ds(base_o + b, stripe)])

    @pl.when(s == NS - 1)
    def _():
        pltpu.sync_copy(acc_sh.at[pl.ds(last, rest)],
                        out_hbm.at[pl.ds(base_o + last, rest)])


def _segsum(edge_attr, receivers, zeros):
    e, h = edge_attr.shape
    n = zeros.shape[0]
    ew = e // NW
    nch = ew // CHUNK
    stripe = (n // NS) // 8 * 8
    mesh = plsc.VectorSubcoreMesh(core_axis_name="c", subcore_axis_name="s")
    k = pl.kernel(
        functools.partial(_segsum_body, n, ew, nch, stripe),
        out_type=jax.ShapeDtypeStruct((NC * n, h), jnp.float32),
        mesh=mesh,
        scratch_types=(
            [pltpu.VMEM((CHUNK,), jnp.int32)] * 4
            + [pltpu.VMEM((CHUNK, h), jnp.float32)] * 4
            + [pltpu.VMEM_SHARED((n, h), jnp.float32)]
            + [pltpu.SemaphoreType.DMA] * 8
        ),
    )
    return k(edge_attr, receivers, zeros)


def _gather_body(ew, nch,
                 xa_hbm, xb_hbm, snd_hbm, rcv_hbm, g_hbm,
                 idxs_all, idxr_all,
                 bufa0, bufb0, bufa1, bufb1, bufa2, bufb2, bufa3, bufb3,
                 gsem0, gsem1, gsem2, gsem3, wsem0, wsem1, wsem2, wsem3):
    c = lax.axis_index("c")
    s = lax.axis_index("s")
    wid = c * NS + s
    base_e = pl.multiple_of(wid * ew, CHUNK)

    bufs = [(bufa0, bufb0, gsem0, wsem0), (bufa1, bufb1, gsem1, wsem1),
            (bufa2, bufb2, gsem2, wsem2), (bufa3, bufb3, gsem3, wsem3)]

    pltpu.sync_copy(snd_hbm.at[pl.ds(base_e, ew)], idxs_all)
    pltpu.sync_copy(rcv_hbm.at[pl.ds(base_e, ew)], idxr_all)

    def islice(ref, i):
        return ref.at[pl.ds(pl.multiple_of(i * CHUNK, CHUNK), CHUNK)]

    def issue(i, sl):
        ba, bb, gsem, _ = bufs[sl]
        pltpu.async_copy(xa_hbm.at[islice(idxs_all, i)], ba, gsem)
        pltpu.async_copy(xb_hbm.at[islice(idxr_all, i)], bb, gsem)

    def wait_gather(sl):
        ba, bb, gsem, _ = bufs[sl]
        pltpu.make_async_copy(xa_hbm.at[pl.ds(0, CHUNK)], ba, gsem).wait()
        pltpu.make_async_copy(xb_hbm.at[pl.ds(0, CHUNK)], bb, gsem).wait()

    def add_wb(i, sl):
        ba, bb, _, wsem = bufs[sl]

        def row(j, carry2):
            for kk in range(8):
                plsc.addupdate(ba.at[j, pl.ds(kk * 16, 16)],
                               bb[j, pl.ds(kk * 16, 16)])
            return carry2

        lax.fori_loop(0, CHUNK, row, 0)
        b = pl.multiple_of(base_e + i * CHUNK, CHUNK)
        pltpu.async_copy(ba, g_hbm.at[pl.ds(b, CHUNK)], wsem)

    def wait_wb(sl):
        ba, _, _, wsem = bufs[sl]
        pltpu.make_async_copy(ba, g_hbm.at[pl.ds(0, CHUNK)], wsem).wait()

    issue(0, 0)
    issue(1, 1)

    def group(j, carry):
        for s4 in range(4):
            cc = 4 * j + s4

            @pl.when(cc + 2 < nch)
            def _():
                @pl.when(cc >= 2)
                def _():
                    wait_wb((s4 + 2) % 4)
                issue(cc + 2, (s4 + 2) % 4)

            wait_gather(s4)
            add_wb(cc, s4)
        return carry

    lax.fori_loop(0, nch // 4, group, 0)
    for s4 in range(nch % 4):
        cc = (nch // 4) * 4 + s4
        wait_gather(s4)
        add_wb(cc, s4)
    for s4 in range(4):
        wait_wb(s4)


def _gather_add(xa, xb, senders, receivers):
    n, h = xa.shape
    e = senders.shape[0]
    ew = e // NW
    nch = ew // CHUNK
    mesh = plsc.VectorSubcoreMesh(core_axis_name="c", subcore_axis_name="s")
    k = pl.kernel(
        functools.partial(_gather_body, ew, nch),
        out_type=jax.ShapeDtypeStruct((e, h), jnp.float32),
        mesh=mesh,
        scratch_types=(
            [pltpu.VMEM((ew,), jnp.int32)] * 2
            + [pltpu.VMEM((CHUNK, h), jnp.float32)] * 8
            + [pltpu.SemaphoreType.DMA] * 8
        ),
    )
    return k(xa, xb, senders, receivers)


def _layer_norm(hh, gamma, beta):
    mu = jnp.mean(hh, axis=-1, keepdims=True)
    var = jnp.mean((hh - mu) ** 2, axis=-1, keepdims=True)
    return (hh - mu) * lax.rsqrt(var + 1e-5) * gamma + beta


def _cell_body(x_ref, na_ref, agg0_ref, agg1_ref,
               w1a_ref, w1b_ref, b1_ref, w2_ref, b2_ref, w3_ref, b3_ref,
               g_ref, bt_ref, ew1a_ref, ew1b_ref,
               xout_ref, xa_ref, xb_ref):
    f32 = jnp.float32
    agg = agg0_ref[...] + agg1_ref[...]
    nb = jnp.dot(na_ref[...], w1b_ref[...], preferred_element_type=f32) + b1_ref[...]

    def mlp(xin):
        hh = jax.nn.silu(
            jnp.dot(xin + agg, w1a_ref[...], preferred_element_type=f32) + nb)
        hh = jax.nn.silu(
            jnp.dot(hh, w2_ref[...], preferred_element_type=f32) + b2_ref[...])
        hh = jnp.dot(hh, w3_ref[...], preferred_element_type=f32) + b3_ref[...]
        return _layer_norm(hh, g_ref[...], bt_ref[...])

    x0 = x_ref[...]
    x2 = mlp(mlp(x0))
    xout_ref[...] = x0 + x2
    xa_ref[...] = jnp.dot(x2, ew1a_ref[...], preferred_element_type=f32)
    xb_ref[...] = jnp.dot(x2, ew1b_ref[...], preferred_element_type=f32)


def _cell(x, node_attr, aggp, cb_W1, cb_b1, cb_W2, cb_b2, cb_W3, cb_b3,
          cb_g, cb_bt, eb_W1):
    n, h = x.shape
    hn = node_attr.shape[1]
    w1a, w1b = cb_W1[:h], cb_W1[h:]
    ew1a, ew1b = eb_W1[:h], eb_W1[h:2 * h]
    row = lambda v: v.reshape(1, h)
    r = 2000
    blk = pl.BlockSpec((r, h), lambda i: (i, 0))
    nblk = pl.BlockSpec((r, hn), lambda i: (i, 0))
    a1blk = pl.BlockSpec((r, h), lambda i: (i + n // r, 0))
    wspec = lambda rows: pl.BlockSpec((rows, h), lambda i: (0, 0))
    out = pl.pallas_call(
        _cell_body,
        grid=(n // r,),
        in_specs=[blk, nblk, blk, a1blk, wspec(h), wspec(hn), wspec(1),
                  wspec(h), wspec(1), wspec(h), wspec(1), wspec(1), wspec(1),
                  wspec(h), wspec(h)],
        out_specs=[blk] * 3,
        out_shape=[jax.ShapeDtypeStruct((n, h), jnp.float32)] * 3,
    )(x, node_attr, aggp, aggp, w1a, w1b, row(cb_b1), cb_W2, row(cb_b2),
      cb_W3, row(cb_b3), row(cb_g), row(cb_bt), ew1a, ew1b)
    return out


def _edge_body(g_ref, e_ref, w1c_ref, b1_ref, w2_ref, b2_ref, w3_ref, b3_ref,
               gm_ref, bt_ref, out_ref):
    f32 = jnp.float32
    e0 = e_ref[...]
    hh = jax.nn.silu(
        g_ref[...]
        + jnp.dot(e0.astype(jnp.bfloat16), w1c_ref[...].astype(jnp.bfloat16),
                  preferred_element_type=f32)
        + b1_ref[...])
    hh = jax.nn.silu(
        jnp.dot(hh.astype(jnp.bfloat16), w2_ref[...].astype(jnp.bfloat16),
                preferred_element_type=f32) + b2_ref[...])
    hh = jnp.dot(hh.astype(jnp.bfloat16), w3_ref[...].astype(jnp.bfloat16),
                 preferred_element_type=f32) + b3_ref[...]
    out_ref[...] = e0 + _layer_norm(hh, gm_ref[...], bt_ref[...])


def _edge(g, edge_attr, eb_W1, eb_b1, eb_W2, eb_b2, eb_W3, eb_b3, eb_g, eb_bt):
    e, h = edge_attr.shape
    r = 8000
    w1c = eb_W1[2 * h:]
    row = lambda v: v.reshape(1, h)
    blk = pl.BlockSpec((r, h), lambda i: (i, 0))
    wspec = pl.BlockSpec((h, h), lambda i: (0, 0))
    bspec = pl.BlockSpec((1, h), lambda i: (0, 0))
    return pl.pallas_call(
        _edge_body,
        grid=(e // r,),
        in_specs=[blk, blk, wspec, bspec, wspec, bspec, wspec, bspec,
                  bspec, bspec],
        out_specs=blk,
        out_shape=jax.ShapeDtypeStruct((e, h), jnp.float32),
    )(g, edge_attr, w1c, row(eb_b1), eb_W2, row(eb_b2), eb_W3, row(eb_b3),
      row(eb_g), row(eb_bt))


def kernel(x, edge_attr, node_attr, edge_index,
           cb_W1, cb_b1, cb_W2, cb_b2, cb_W3, cb_b3, cb_g, cb_bt,
           eb_W1, eb_b1, eb_W2, eb_b2, eb_W3, eb_b3, eb_g, eb_bt):
    senders = edge_index[0]
    receivers = edge_index[1]
    n, h = x.shape
    zeros = jnp.zeros((n, h), jnp.float32)
    aggp = _segsum(edge_attr, receivers, zeros)
    x_out, xa, xb = _cell(x, node_attr, aggp, cb_W1, cb_b1, cb_W2, cb_b2,
                          cb_W3, cb_b3, cb_g, cb_bt, eb_W1)
    g = _gather_add(xa, xb, senders, receivers)
    e_out = _edge(g, edge_attr, eb_W1, eb_b1, eb_W2, eb_b2, eb_W3, eb_b3,
                  eb_g, eb_bt)
    return (x_out, e_out)


# speedup vs baseline: 1.4184x; 1.0005x over previous
import functools

import jax
import jax.numpy as jnp
from jax import lax
from jax.experimental import pallas as pl
from jax.experimental.pallas import tpu as pltpu
from jax.experimental.pallas import tpu_sc as plsc

NC = 2
NS = 16
NW = NC * NS
CHUNK = 80


def _segsum_body(n, ew, nch, stripe,
                 edge_hbm, recv_hbm, zeros_hbm, out_hbm,
                 idx0, idx1, idx2, idx3, rows0, rows1, rows2, rows3, acc_sh,
                 lsem0, lsem1, lsem2, lsem3, ssem0, ssem1, ssem2, ssem3):
    c = lax.axis_index("c")
    s = lax.axis_index("s")
    wid = c * NS + s
    last = stripe * (NS - 1)
    rest = n - last

    @pl.when(s < NS - 1)
    def _():
        b = pl.multiple_of(s * stripe, 8)
        pltpu.sync_copy(zeros_hbm.at[pl.ds(b, stripe)],
                        acc_sh.at[pl.ds(b, stripe)])

    @pl.when(s == NS - 1)
    def _():
        pltpu.sync_copy(zeros_hbm.at[pl.ds(last, rest)],
                        acc_sh.at[pl.ds(last, rest)])

    plsc.subcore_barrier()
    base_e = pl.multiple_of(wid * ew, CHUNK)
    rows = [(idx0, rows0, lsem0, ssem0), (idx1, rows1, lsem1, ssem1),
            (idx2, rows2, lsem2, ssem2), (idx3, rows3, lsem3, ssem3)]

    def load(i, sl):
        iv, rv, lsem, _ = rows[sl]
        b = pl.multiple_of(base_e + i * CHUNK, CHUNK)
        pltpu.async_copy(recv_hbm.at[pl.ds(b, CHUNK)], iv, lsem)
        pltpu.async_copy(edge_hbm.at[pl.ds(b, CHUNK)], rv, lsem)

    def wait_load(sl):
        iv, rv, lsem, _ = rows[sl]
        pltpu.make_async_copy(recv_hbm.at[pl.ds(0, CHUNK)], iv, lsem).wait()
        pltpu.make_async_copy(edge_hbm.at[pl.ds(0, CHUNK)], rv, lsem).wait()

    def scatter(i, sl):
        iv, rv, _, ssem = rows[sl]
        pltpu.async_copy(rv, acc_sh.at[iv], ssem, add=True)

    def wait_scatter(sl):
        iv, rv, _, ssem = rows[sl]
        pltpu.make_async_copy(rv, acc_sh.at[iv], ssem).wait()

    load(0, 0)
    load(1, 1)

    def group(j, carry):
        for s4 in range(4):
            cc = 4 * j + s4

            @pl.when(cc + 2 < nch)
            def _():
                @pl.when(cc >= 2)
                def _():
                    wait_scatter((s4 + 2) % 4)
                load(cc + 2, (s4 + 2) % 4)

            wait_load(s4)
            scatter(cc, s4)
        return carry

    lax.fori_loop(0, nch // 4, group, 0)
    for s4 in range(nch % 4):
        cc = (nch // 4) * 4 + s4
        wait_load(s4)
        scatter(cc, s4)
    for s4 in range(4):
        wait_scatter(s4)
    plsc.subcore_barrier()

    base_o = pl.multiple_of(c * n, 8)

    @pl.when(s < NS - 1)
    def _():
        b = pl.multiple_of(s * stripe, 8)
        pltpu.sync_copy(acc_sh.at[pl.ds(b, stripe)],
                        out_hbm.at[pl.ds(base_o + b, stripe)])

    @pl.when(s == NS - 1)
    def _():
        pltpu.sync_copy(acc_sh.at[pl.ds(last, rest)],
                        out_hbm.at[pl.ds(base_o + last, rest)])


def _segsum(edge_attr, receivers, zeros):
    e, h = edge_attr.shape
    n = zeros.shape[0]
    ew = e // NW
    nch = ew // CHUNK
    stripe = (n // NS) // 8 * 8
    mesh = plsc.VectorSubcoreMesh(core_axis_name="c", subcore_axis_name="s")
    k = pl.kernel(
        functools.partial(_segsum_body, n, ew, nch, stripe),
        out_type=jax.ShapeDtypeStruct((NC * n, h), jnp.float32),
        mesh=mesh,
        scratch_types=(
            [pltpu.VMEM((CHUNK,), jnp.int32)] * 4
            + [pltpu.VMEM((CHUNK, h), jnp.float32)] * 4
            + [pltpu.VMEM_SHARED((n, h), jnp.float32)]
            + [pltpu.SemaphoreType.DMA] * 8
        ),
    )
    return k(edge_attr, receivers, zeros)


def _gather_body(ew, nch,
                 xa_hbm, xb_hbm, snd_hbm, rcv_hbm, g_hbm,
                 idxs_all, idxr_all,
                 bufa0, bufb0, bufa1, bufb1, bufa2, bufb2, bufa3, bufb3,
                 gsem0, gsem1, gsem2, gsem3, wsem0, wsem1, wsem2, wsem3):
    c = lax.axis_index("c")
    s = lax.axis_index("s")
    wid = c * NS + s
    base_e = pl.multiple_of(wid * ew, CHUNK)

    bufs = [(bufa0, bufb0, gsem0, wsem0), (bufa1, bufb1, gsem1, wsem1),
            (bufa2, bufb2, gsem2, wsem2), (bufa3, bufb3, gsem3, wsem3)]

    pltpu.sync_copy(snd_hbm.at[pl.ds(base_e, ew)], idxs_all)
    pltpu.sync_copy(rcv_hbm.at[pl.ds(base_e, ew)], idxr_all)

    def islice(ref, i):
        return ref.at[pl.ds(pl.multiple_of(i * CHUNK, CHUNK), CHUNK)]

    def issue(i, sl):
        ba, bb, gsem, _ = bufs[sl]
        pltpu.async_copy(xa_hbm.at[islice(idxs_all, i)], ba, gsem)
        pltpu.async_copy(xb_hbm.at[islice(idxr_all, i)], bb, gsem)

    def wait_gather(sl):
        ba, bb, gsem, _ = bufs[sl]
        pltpu.make_async_copy(xa_hbm.at[pl.ds(0, CHUNK)], ba, gsem).wait()
        pltpu.make_async_copy(xb_hbm.at[pl.ds(0, CHUNK)], bb, gsem).wait()

    def add_wb(i, sl):
        ba, bb, _, wsem = bufs[sl]

        def row(j, carry2):
            for kk in range(8):
                plsc.addupdate(ba.at[j, pl.ds(kk * 16, 16)],
                               bb[j, pl.ds(kk * 16, 16)])
            return carry2

        lax.fori_loop(0, CHUNK, row, 0)
        b = pl.multiple_of(base_e + i * CHUNK, CHUNK)
        pltpu.async_copy(ba, g_hbm.at[pl.ds(b, CHUNK)], wsem)

    def wait_wb(sl):
        ba, _, _, wsem = bufs[sl]
        pltpu.make_async_copy(ba, g_hbm.at[pl.ds(0, CHUNK)], wsem).wait()

    issue(0, 0)
    issue(1, 1)

    def group(j, carry):
        for s4 in range(4):
            cc = 4 * j + s4

            @pl.when(cc + 2 < nch)
            def _():
                @pl.when(cc >= 2)
                def _():
                    wait_wb((s4 + 2) % 4)
                issue(cc + 2, (s4 + 2) % 4)

            wait_gather(s4)
            add_wb(cc, s4)
        return carry

    lax.fori_loop(0, nch // 4, group, 0)
    for s4 in range(nch % 4):
        cc = (nch // 4) * 4 + s4
        wait_gather(s4)
        add_wb(cc, s4)
    for s4 in range(4):
        wait_wb(s4)


def _gather_add(xa, xb, senders, receivers):
    n, h = xa.shape
    e = senders.shape[0]
    ew = e // NW
    nch = ew // CHUNK
    mesh = plsc.VectorSubcoreMesh(core_axis_name="c", subcore_axis_name="s")
    k = pl.kernel(
        functools.partial(_gather_body, ew, nch),
        out_type=jax.ShapeDtypeStruct((e, h), jnp.float32),
        mesh=mesh,
        scratch_types=(
            [pltpu.VMEM((ew,), jnp.int32)] * 2
            + [pltpu.VMEM((CHUNK, h), jnp.float32)] * 8
            + [pltpu.SemaphoreType.DMA] * 8
        ),
    )
    return k(xa, xb, senders, receivers)


def _layer_norm(hh, gamma, beta):
    mu = jnp.mean(hh, axis=-1, keepdims=True)
    var = jnp.mean((hh - mu) ** 2, axis=-1, keepdims=True)
    return (hh - mu) * lax.rsqrt(var + 1e-5) * gamma + beta


def _cell_body(x_ref, na_ref, agg0_ref, agg1_ref,
               w1a_ref, w1b_ref, b1_ref, w2_ref, b2_ref, w3_ref, b3_ref,
               g_ref, bt_ref, ew1a_ref, ew1b_ref,
               xout_ref, xa_ref, xb_ref):
    f32 = jnp.float32
    agg = agg0_ref[...] + agg1_ref[...]
    nb = jnp.dot(na_ref[...], w1b_ref[...], preferred_element_type=f32) + b1_ref[...]

    def mlp(xin):
        hh = jax.nn.silu(
            jnp.dot(xin + agg, w1a_ref[...], preferred_element_type=f32) + nb)
        hh = jax.nn.silu(
            jnp.dot(hh, w2_ref[...], preferred_element_type=f32) + b2_ref[...])
        hh = jnp.dot(hh, w3_ref[...], preferred_element_type=f32) + b3_ref[...]
        return _layer_norm(hh, g_ref[...], bt_ref[...])

    x0 = x_ref[...]
    x2 = mlp(mlp(x0))
    xout_ref[...] = x0 + x2
    xa_ref[...] = jnp.dot(x2, ew1a_ref[...], preferred_element_type=f32)
    xb_ref[...] = jnp.dot(x2, ew1b_ref[...], preferred_element_type=f32)


def _cell(x, node_attr, aggp, cb_W1, cb_b1, cb_W2, cb_b2, cb_W3, cb_b3,
          cb_g, cb_bt, eb_W1):
    n, h = x.shape
    hn = node_attr.shape[1]
    w1a, w1b = cb_W1[:h], cb_W1[h:]
    ew1a, ew1b = eb_W1[:h], eb_W1[h:2 * h]
    row = lambda v: v.reshape(1, h)
    r = 2000
    blk = pl.BlockSpec((r, h), lambda i: (i, 0))
    nblk = pl.BlockSpec((r, hn), lambda i: (i, 0))
    a1blk = pl.BlockSpec((r, h), lambda i: (i + n // r, 0))
    wspec = lambda rows: pl.BlockSpec((rows, h), lambda i: (0, 0))
    out = pl.pallas_call(
        _cell_body,
        grid=(n // r,),
        in_specs=[blk, nblk, blk, a1blk, wspec(h), wspec(hn), wspec(1),
                  wspec(h), wspec(1), wspec(h), wspec(1), wspec(1), wspec(1),
                  wspec(h), wspec(h)],
        out_specs=[blk] * 3,
        out_shape=[jax.ShapeDtypeStruct((n, h), jnp.float32)] * 3,
    )(x, node_attr, aggp, aggp, w1a, w1b, row(cb_b1), cb_W2, row(cb_b2),
      cb_W3, row(cb_b3), row(cb_g), row(cb_bt), ew1a, ew1b)
    return out


def _edge_body(g_ref, e_ref, w1c_ref, b1_ref, w2_ref, b2_ref, w3_ref, b3_ref,
               gm_ref, bt_ref, out_ref):
    f32 = jnp.float32
    e0 = e_ref[...]
    hh = jax.nn.silu(
        g_ref[...] + jnp.dot(e0, w1c_ref[...], preferred_element_type=f32)
        + b1_ref[...])
    hh = jax.nn.silu(
        jnp.dot(hh, w2_ref[...], preferred_element_type=f32) + b2_ref[...])
    hh = jnp.dot(hh, w3_ref[...], preferred_element_type=f32) + b3_ref[...]
    out_ref[...] = e0 + _layer_norm(hh, gm_ref[...], bt_ref[...])


def _edge(g, edge_attr, eb_W1, eb_b1, eb_W2, eb_b2, eb_W3, eb_b3, eb_g, eb_bt):
    e, h = edge_attr.shape
    r = 8000
    w1c = eb_W1[2 * h:]
    row = lambda v: v.reshape(1, h)
    blk = pl.BlockSpec((r, h), lambda i: (i, 0))
    wspec = pl.BlockSpec((h, h), lambda i: (0, 0))
    bspec = pl.BlockSpec((1, h), lambda i: (0, 0))
    return pl.pallas_call(
        _edge_body,
        grid=(e // r,),
        in_specs=[blk, blk, wspec, bspec, wspec, bspec, wspec, bspec,
                  bspec, bspec],
        out_specs=blk,
        out_shape=jax.ShapeDtypeStruct((e, h), jnp.float32),
    )(g, edge_attr, w1c, row(eb_b1), eb_W2, row(eb_b2), eb_W3, row(eb_b3),
      row(eb_g), row(eb_bt))


def kernel(x, edge_attr, node_attr, edge_index,
           cb_W1, cb_b1, cb_W2, cb_b2, cb_W3, cb_b3, cb_g, cb_bt,
           eb_W1, eb_b1, eb_W2, eb_b2, eb_W3, eb_b3, eb_g, eb_bt):
    senders = edge_index[0]
    receivers = edge_index[1]
    n, h = x.shape
    zeros = jnp.zeros((n, h), jnp.float32)
    aggp = _segsum(edge_attr, receivers, zeros)
    x_out, xa, xb = _cell(x, node_attr, aggp, cb_W1, cb_b1, cb_W2, cb_b2,
                          cb_W3, cb_b3, cb_g, cb_bt, eb_W1)
    g = _gather_add(xa, xb, senders, receivers)
    e_out = _edge(g, edge_attr, eb_W1, eb_b1, eb_W2, eb_b2, eb_W3, eb_b3,
                  eb_g, eb_bt)
    return (x_out, e_out)
